# baseline (speedup 1.0000x reference)
import jax
import jax.numpy as jnp
from jax.experimental import pallas as pl

D_MODEL = 4096
NUM_EXPERTS = 64
TOP_K = 8
Z_LOSS = 0.001

BT = 1024


def _router_block(x_ref, w_ref, idx_ref, wts_ref, counts_ref, stats_ref, fin_ref):
    step = pl.program_id(0)
    nsteps = pl.num_programs(0)

    h = x_ref[...]
    w = w_ref[...]
    logits = jax.lax.dot_general(
        h, w, (((1,), (1,)), ((), ())), preferred_element_type=jnp.float32
    )

    coli = jax.lax.broadcasted_iota(jnp.int32, logits.shape, 1)
    u = jax.lax.bitcast_convert_type(logits, jnp.int32)
    low6 = jnp.where(u < 0, coli, jnp.int32(63) - coli)
    kcur = jax.lax.bitcast_convert_type((u & jnp.int32(-64)) | low6, jnp.float32)
    keys = []
    for _ in range(TOP_K):
        vk = jnp.max(kcur, axis=-1)
        keys.append(vk)
        kcur = jnp.where(kcur == vk[:, None], -jnp.inf, kcur)

    m = jax.lax.bitcast_convert_type(
        jax.lax.bitcast_convert_type(keys[0], jnp.int32) & jnp.int32(-64),
        jnp.float32,
    )
    lm = logits - m[:, None]
    e = jnp.exp(lm)
    s = jnp.sum(e, axis=-1)
    logs = jnp.log(s)
    ent = logs - jnp.sum(e * lm, axis=-1) / s
    z = m + logs

    inv_s = 1.0 / s
    tis = []
    ps = []
    for k in range(TOP_K):
        kb = jax.lax.bitcast_convert_type(keys[k], jnp.int32)
        low = kb & jnp.int32(63)
        tis.append(jnp.where(kb < 0, low, jnp.int32(63) - low))
        tv = jax.lax.bitcast_convert_type(kb & jnp.int32(-64), jnp.float32)
        ps.append(jnp.exp(tv - m) * inv_s)
    psum = ps[0]
    for k in range(1, TOP_K):
        psum = psum + ps[k]
    r = 1.0 / (psum + 1e-9)
    ws = [p * r for p in ps]

    idx_ref[...] = jnp.stack(tis, axis=0)
    wts_ref[...] = jnp.stack(ws, axis=0)

    onehot = (coli == tis[0][:, None]).astype(jnp.float32)
    ones_row = jnp.ones((1, BT), jnp.float32)
    cnt = jax.lax.dot_general(
        ones_row, onehot, (((1,), (0,)), ((), ())),
        preferred_element_type=jnp.float32,
    )

    part = jnp.stack(
        [
            jnp.sum(ent),
            jnp.min(ent),
            jnp.sum(z * z),
            jnp.sum(logits * logits),
            jnp.sum(ws[0] - ws[1]),
            jnp.sum(ws[0]),
            0.0,
            0.0,
        ]
    )[None, :]

    @pl.when(step == 0)
    def _():
        counts_ref[...] = cnt
        stats_ref[...] = part

    @pl.when(step != 0)
    def _():
        counts_ref[...] += cnt
        old = stats_ref[...]
        lane = jax.lax.broadcasted_iota(jnp.int32, old.shape, 1)
        stats_ref[...] = jnp.where(lane == 1, jnp.minimum(old, part), old + part)

    @pl.when(step == nsteps - 1)
    def _():
        t_tot = jnp.float32(nsteps * BT)
        counts = counts_ref[0, :]
        stats = stats_ref[0, :]
        cmean = jnp.sum(counts) / NUM_EXPERTS
        cstd = jnp.sqrt(jnp.sum((counts - cmean) ** 2) / NUM_EXPERTS)
        cv = cstd / (cmean + 1e-9)
        fin_ref[...] = jnp.stack(
            [
                stats[0] / t_tot,
                stats[1],
                cv,
                Z_LOSS * stats[2] / t_tot,
                jnp.sqrt(stats[3] / (t_tot * NUM_EXPERTS)),
                stats[4] / t_tot,
                stats[5] / t_tot,
                0.0,
            ]
        )[None, :]


def kernel(x, W):
    B, S, D = x.shape
    T = B * S
    h = x.reshape(T, D)
    nsteps = T // BT

    topi_t, topw_t, counts, _, fin = pl.pallas_call(
        _router_block,
        grid=(nsteps,),
        in_specs=[
            pl.BlockSpec((BT, D), lambda i: (i, 0)),
            pl.BlockSpec((NUM_EXPERTS, D), lambda i: (0, 0)),
        ],
        out_specs=[
            pl.BlockSpec((TOP_K, BT), lambda i: (0, i)),
            pl.BlockSpec((TOP_K, BT), lambda i: (0, i)),
            pl.BlockSpec((1, NUM_EXPERTS), lambda i: (0, 0)),
            pl.BlockSpec((1, 8), lambda i: (0, 0)),
            pl.BlockSpec((1, 8), lambda i: (0, 0)),
        ],
        out_shape=[
            jax.ShapeDtypeStruct((TOP_K, T), jnp.int32),
            jax.ShapeDtypeStruct((TOP_K, T), jnp.float32),
            jax.ShapeDtypeStruct((1, NUM_EXPERTS), jnp.float32),
            jax.ShapeDtypeStruct((1, 8), jnp.float32),
            jax.ShapeDtypeStruct((1, 8), jnp.float32),
        ],
    )(h, W)

    return (
        topi_t.T.astype(jnp.int64),
        topw_t.T,
        fin[0, 0],
        fin[0, 1],
        fin[0, 2],
        counts[0],
        fin[0, 3],
        fin[0, 4],
        fin[0, 5],
        fin[0, 6],
    )


# speedup vs baseline: 1.0042x; 1.0042x over previous
import jax
import jax.numpy as jnp
from jax.experimental import pallas as pl

D_MODEL = 4096
NUM_EXPERTS = 64
TOP_K = 8
Z_LOSS = 0.001

BT = 1024


def _router_block(x_ref, w_ref, idx_ref, wts_ref, counts_ref, stats_ref, fin_ref):
    step = pl.program_id(0)
    nsteps = pl.num_programs(0)

    h = x_ref[...]
    w = w_ref[...]
    logits = jax.lax.dot_general(
        h, w, (((1,), (1,)), ((), ())), preferred_element_type=jnp.float32
    )

    coli = jax.lax.broadcasted_iota(jnp.int32, logits.shape, 1)
    u = jax.lax.bitcast_convert_type(logits, jnp.int32)
    low6 = jnp.where(u < 0, coli, jnp.int32(63) - coli)
    kcur = jax.lax.bitcast_convert_type((u & jnp.int32(-64)) | low6, jnp.float32)
    keys = []
    for _ in range(TOP_K):
        vk = jnp.max(kcur, axis=-1)
        keys.append(vk)
        kcur = jnp.where(kcur == vk[:, None], -jnp.inf, kcur)

    m = jax.lax.bitcast_convert_type(
        jax.lax.bitcast_convert_type(keys[0], jnp.int32) & jnp.int32(-64),
        jnp.float32,
    )
    lm = logits - m[:, None]
    e = jnp.exp(lm)
    s = jnp.sum(e, axis=-1)
    logs = jnp.log(s)
    ent = logs - jnp.sum(e * lm, axis=-1) / s
    z = m + logs

    inv_s = 1.0 / s
    tis = []
    ps = []
    for k in range(TOP_K):
        kb = jax.lax.bitcast_convert_type(keys[k], jnp.int32)
        low = kb & jnp.int32(63)
        tis.append(jnp.where(kb < 0, low, jnp.int32(63) - low))
        tv = jax.lax.bitcast_convert_type(kb & jnp.int32(-64), jnp.float32)
        ps.append(jnp.exp(tv - m) * inv_s)
    psum = ps[0]
    for k in range(1, TOP_K):
        psum = psum + ps[k]
    r = 1.0 / (psum + 1e-9)
    ws = [p * r for p in ps]

    idx_ref[...] = jnp.stack(tis, axis=0)
    wts_ref[...] = jnp.stack(ws, axis=0)

    onehot = (coli == tis[0][:, None]).astype(jnp.float32)
    cnt = jnp.sum(onehot, axis=0)[None, :]

    part = jnp.stack(
        [
            jnp.sum(ent),
            jnp.min(ent),
            jnp.sum(z * z),
            jnp.sum(logits * logits),
            jnp.sum(ws[0] - ws[1]),
            jnp.sum(ws[0]),
            0.0,
            0.0,
        ]
    )[None, :]

    @pl.when(step == 0)
    def _():
        counts_ref[...] = cnt
        stats_ref[...] = part

    @pl.when(step != 0)
    def _():
        counts_ref[...] += cnt
        old = stats_ref[...]
        lane = jax.lax.broadcasted_iota(jnp.int32, old.shape, 1)
        stats_ref[...] = jnp.where(lane == 1, jnp.minimum(old, part), old + part)

    @pl.when(step == nsteps - 1)
    def _():
        t_tot = jnp.float32(nsteps * BT)
        counts = counts_ref[0, :]
        stats = stats_ref[0, :]
        cmean = jnp.sum(counts) / NUM_EXPERTS
        cstd = jnp.sqrt(jnp.sum((counts - cmean) ** 2) / NUM_EXPERTS)
        cv = cstd / (cmean + 1e-9)
        fin_ref[...] = jnp.stack(
            [
                stats[0] / t_tot,
                stats[1],
                cv,
                Z_LOSS * stats[2] / t_tot,
                jnp.sqrt(stats[3] / (t_tot * NUM_EXPERTS)),
                stats[4] / t_tot,
                stats[5] / t_tot,
                0.0,
            ]
        )[None, :]


def kernel(x, W):
    B, S, D = x.shape
    T = B * S
    h = x.reshape(T, D)
    nsteps = T // BT

    topi_t, topw_t, counts, _, fin = pl.pallas_call(
        _router_block,
        grid=(nsteps,),
        in_specs=[
            pl.BlockSpec((BT, D), lambda i: (i, 0)),
            pl.BlockSpec((NUM_EXPERTS, D), lambda i: (0, 0)),
        ],
        out_specs=[
            pl.BlockSpec((TOP_K, BT), lambda i: (0, i)),
            pl.BlockSpec((TOP_K, BT), lambda i: (0, i)),
            pl.BlockSpec((1, NUM_EXPERTS), lambda i: (0, 0)),
            pl.BlockSpec((1, 8), lambda i: (0, 0)),
            pl.BlockSpec((1, 8), lambda i: (0, 0)),
        ],
        out_shape=[
            jax.ShapeDtypeStruct((TOP_K, T), jnp.int32),
            jax.ShapeDtypeStruct((TOP_K, T), jnp.float32),
            jax.ShapeDtypeStruct((1, NUM_EXPERTS), jnp.float32),
            jax.ShapeDtypeStruct((1, 8), jnp.float32),
            jax.ShapeDtypeStruct((1, 8), jnp.float32),
        ],
    )(h, W)

    return (
        topi_t.T.astype(jnp.int64),
        topw_t.T,
        fin[0, 0],
        fin[0, 1],
        fin[0, 2],
        counts[0],
        fin[0, 3],
        fin[0, 4],
        fin[0, 5],
        fin[0, 6],
    )
